# baseline (speedup 1.0000x reference)
import functools
import math

import jax
import jax.numpy as jnp
from jax import lax
from jax.experimental import pallas as pl
from jax.experimental.pallas import tpu as pltpu
from jax.experimental.pallas import tpu_sc as plsc

N = 10000
E = 320000
IN_DIM = 128
HIDDEN = 128
DEPTH = 3
TOTAL_BLOCKS = 7
POS_DIM = 3
WIDTH = 512

NE = E + N
CHUNK = 64
NTILES = 16
NE_PAD = ((NE + NTILES * CHUNK - 1) // (NTILES * CHUNK)) * (NTILES * CHUNK)
CH_PER_TILE = NE_PAD // (NTILES * CHUNK)
NPASS = 2
STRIDE = 5056
ACC_ROWS = 5120
RPT = ACC_ROWS // 16
TRASH = ACC_ROWS - 1

BR = 2000
GRID = N // BR


def _make_sc_spmm(d):
    mesh = plsc.VectorSubcoreMesh(core_axis_name="c", subcore_axis_name="s")

    @functools.partial(
        pl.kernel,
        out_type=jax.ShapeDtypeStruct((NPASS, ACC_ROWS, d), jnp.float32),
        mesh=mesh,
        scratch_types=[
            pltpu.VMEM((3, CHUNK), jnp.int32),
            pltpu.VMEM((3, CHUNK), jnp.int32),
            pltpu.VMEM((2, CHUNK, d), jnp.float32),
            pltpu.VMEM_SHARED((ACC_ROWS, d), jnp.float32),
            pltpu.SemaphoreType.DMA,
            pltpu.SemaphoreType.DMA,
        ],
    )
    def spmm(z_hbm, srcs_hbm, dsts_hbm, out_hbm,
             sidx, didx, rbuf, acc, isem, gsem):
        wid = lax.axis_index("s")
        p = lax.axis_index("c")
        r0 = wid * RPT
        zv = jnp.zeros((16,), jnp.float32)

        def idx_start(p, j, slot):
            pltpu.async_copy(srcs_hbm.at[p].at[wid].at[j], sidx.at[slot], isem)
            pltpu.async_copy(dsts_hbm.at[p].at[wid].at[j], didx.at[slot], isem)

        def idx_wait(p, j, slot):
            pltpu.make_async_copy(srcs_hbm.at[p].at[wid].at[j], sidx.at[slot],
                                  isem).wait()
            pltpu.make_async_copy(dsts_hbm.at[p].at[wid].at[j], didx.at[slot],
                                  isem).wait()

        if True:
            @pl.loop(0, CHUNK)
            def _(r):
                @pl.loop(0, d // 16)
                def _(cc):
                    rbuf.at[0].at[r][pl.ds(cc * 16, 16)] = zv

            @pl.loop(0, RPT // CHUNK)
            def _(i):
                pltpu.sync_copy(rbuf.at[0],
                                acc.at[pl.ds(r0 + i * CHUNK, CHUNK)])
            plsc.subcore_barrier()

            idx_start(p, 0, 0)
            idx_start(p, 1, 1)
            idx_wait(p, 0, 0)
            pltpu.async_copy(z_hbm.at[sidx.at[0]], rbuf.at[0], gsem)

            @pl.loop(0, CH_PER_TILE)
            def _(j):
                b = lax.rem(j, 2)
                nb = 1 - b
                s2 = lax.rem(j + 2, 3)
                s1 = lax.rem(j + 1, 3)

                @pl.when(j + 2 < CH_PER_TILE)
                def _():
                    idx_start(p, j + 2, s2)

                @pl.when(j + 1 < CH_PER_TILE)
                def _():
                    idx_wait(p, j + 1, s1)
                    pltpu.async_copy(z_hbm.at[sidx.at[s1]], rbuf.at[nb], gsem)

                pltpu.make_async_copy(z_hbm.at[sidx.at[lax.rem(j, 3)]],
                                      rbuf.at[b], gsem).wait()
                pltpu.sync_copy(rbuf.at[b], acc.at[didx.at[lax.rem(j, 3)]],
                                add=True)

            plsc.subcore_barrier()
            pltpu.sync_copy(acc.at[pl.ds(r0, RPT)],
                            out_hbm.at[p].at[pl.ds(r0, RPT)])
            plsc.subcore_barrier()

    return spmm


_SC_CACHE = {}


def _sc_spmm(d, *args):
    if d not in _SC_CACHE:
        _SC_CACHE[d] = _make_sc_spmm(d)
    return _SC_CACHE[d](*args)


def _mm(x, w):
    return lax.dot_general(x, w, (((1,), (1,)), ((), ())),
                           preferred_element_type=jnp.float32)


def _ln(z, g, b, eps=1e-5):
    m = jnp.mean(z, axis=-1, keepdims=True)
    v = jnp.mean((z - m) ** 2, axis=-1, keepdims=True)
    return (z - m) / jnp.sqrt(v + eps) * g + b


def _silu(x):
    return x * (1.0 / (1.0 + jnp.exp(-x)))


def _scale_of(deg, m):
    return m * lax.rsqrt(jnp.maximum(deg, 1.0))


def _full(shape):
    return pl.BlockSpec(shape, lambda i: (0, 0))


def _rows(width=HIDDEN):
    return pl.BlockSpec((BR, width), lambda i: (i, 0))


def _prelude_body(cond_ref, f1w, f1b, f2w, f2b, f3w, f3b,
                  c1w, c1b, c2w, c2b, cpw, cpb,
                  f_out, cv_out, cp_out):
    c8 = cond_ref[...]
    t = _silu(_mm(c8, f1w[...]) + f1b[...])
    t = _silu(_mm(t, f2w[...]) + f2b[...])
    f_out[...] = _mm(t, f3w[...]) + f3b[...]
    cv_out[...] = _mm(_silu(_mm(c8, c1w[...]) + c1b[...]), c2w[...]) + c2b[...]
    cp_out[...] = _mm(c8, cpw[...]) + cpb[...]


def _prelude(cond8, p):
    cpw = jnp.concatenate([p['blk%d_cp_W' % i] for i in range(TOTAL_BLOCKS)], 0)
    cpb = jnp.concatenate([p['blk%d_cp_b' % i] for i in range(TOTAL_BLOCKS)], 0)
    nf = TOTAL_BLOCKS * 2 * HIDDEN
    f, cv, cp = pl.pallas_call(
        _prelude_body,
        out_shape=[jax.ShapeDtypeStruct((8, nf), jnp.float32),
                   jax.ShapeDtypeStruct((8, HIDDEN), jnp.float32),
                   jax.ShapeDtypeStruct((8, TOTAL_BLOCKS * HIDDEN), jnp.float32)],
    )(cond8,
      p['film1_W'], p['film1_b'].reshape(1, -1),
      p['film2_W'], p['film2_b'].reshape(1, -1),
      p['film3_W'], p['film3_b'].reshape(1, -1),
      p['cond1_W'], p['cond1_b'].reshape(1, -1),
      p['cond2_W'], p['cond2_b'].reshape(1, -1),
      cpw, cpb.reshape(1, -1))
    f = f[0].reshape(TOTAL_BLOCKS, 2, HIDDEN)
    gammas = 1.0 + f[:, 0, :]
    betas = f[:, 1, :]
    cps = cp[0].reshape(TOTAL_BLOCKS, HIDDEN)
    return gammas, betas, cps, cv[0:1]


def _input_body(x, pos, inw, inb, p1w, p1b, p2w, p2b, cv, out):
    h = _mm(x[...], inw[...]) + inb[...]
    h = h + _mm(_silu(_mm(pos[...], p1w[...]) + p1b[...]), p2w[...]) + p2b[...]
    out[...] = h + cv[...]


def _input_proj(x, pospad, p, cv):
    p1w = jnp.pad(p['pos1_W'], ((0, 0), (0, HIDDEN - POS_DIM)))
    return pl.pallas_call(
        _input_body,
        grid=(GRID,),
        in_specs=[_rows(), _rows(), _full((HIDDEN, IN_DIM)), _full((1, HIDDEN)),
                  _full((HIDDEN, HIDDEN)), _full((1, HIDDEN)),
                  _full((HIDDEN, HIDDEN)), _full((1, HIDDEN)),
                  _full((1, HIDDEN))],
        out_specs=_rows(),
        out_shape=jax.ShapeDtypeStruct((N, HIDDEN), jnp.float32),
    )(x, pospad, p['in_proj_W'], p['in_proj_b'].reshape(1, -1),
      p1w, p['pos1_b'].reshape(1, -1),
      p['pos2_W'], p['pos2_b'].reshape(1, -1), cv)


def _pre_body(h, deg, m, g, b, cp, zs):
    z = _silu(_ln(h[...], g[...], b[...]) + cp[...])
    zs[...] = z * _scale_of(deg[...], m[...])


def _pre(h, deg, m, g, b, cp):
    return pl.pallas_call(
        _pre_body,
        grid=(GRID,),
        in_specs=[_rows(), _rows(1), _rows(1), _full((1, HIDDEN)),
                  _full((1, HIDDEN)), _full((1, HIDDEN))],
        out_specs=_rows(),
        out_shape=jax.ShapeDtypeStruct((N, HIDDEN), jnp.float32),
    )(h, deg, m, g, b, cp)


def _preu_body(h, skip, mn, deg, m, g, b, cp, heff_o, zs):
    heff = jnp.where(mn[...] > 0, h[...], 0.0) + skip[...]
    heff_o[...] = heff
    z = _silu(_ln(heff, g[...], b[...]) + cp[...])
    zs[...] = z * _scale_of(deg[...], m[...])


def _pre_unpool(h, skip, mn, deg, m, g, b, cp):
    return pl.pallas_call(
        _preu_body,
        grid=(GRID,),
        in_specs=[_rows(), _rows(), _rows(1), _rows(1), _rows(1),
                  _full((1, HIDDEN)), _full((1, HIDDEN)), _full((1, HIDDEN))],
        out_specs=[_rows(), _rows()],
        out_shape=[jax.ShapeDtypeStruct((N, HIDDEN), jnp.float32),
                   jax.ShapeDtypeStruct((N, HIDDEN), jnp.float32)],
    )(h, skip, mn, deg, m, g, b, cp)


def _post_body(h, a0, deg, m, cw, cb, g, b, gam, bet, out):
    gsum = a0[...] * _scale_of(deg[...], m[...])
    z = _mm(gsum, cw[...]) + cb[...]
    z = _ln(z, g[...], b[...])
    out[...] = h[...] + z * gam[...] + bet[...]


def _post(h, acc, deg, m, cw, cb, g, b, gam, bet):
    return pl.pallas_call(
        _post_body,
        grid=(GRID,),
        in_specs=[_rows(), _rows(), _rows(1), _rows(1),
                  _full((HIDDEN, HIDDEN)), _full((1, HIDDEN)),
                  _full((1, HIDDEN)), _full((1, HIDDEN)),
                  _full((1, HIDDEN)), _full((1, HIDDEN))],
        out_specs=_rows(),
        out_shape=jax.ShapeDtypeStruct((N, HIDDEN), jnp.float32),
    )(h, acc, deg, m, cw, cb, g, b, gam, bet)


def _score_body(h, m, g, b, s1w, s1b, s2w, s2b, out):
    z = _ln(h[...], g[...], b[...])
    z = _silu(_mm(z, s1w[...]) + s1b[...])
    s = _mm(z, s2w[...]) + s2b[...]
    out[...] = jnp.where(m[...] > 0, s[:, 0:1], -1e30)


def _score(h, m, p, d):
    s2w = jnp.pad(p['pool%d_s2_W' % d], ((0, 7), (0, 0)))
    s2b = jnp.broadcast_to(p['pool%d_s2_b' % d].reshape(1, 1), (1, 8))
    return pl.pallas_call(
        _score_body,
        grid=(GRID,),
        in_specs=[_rows(), _rows(1), _full((1, HIDDEN)), _full((1, HIDDEN)),
                  _full((HIDDEN // 2, HIDDEN)), _full((1, HIDDEN // 2)),
                  _full((8, HIDDEN // 2)), _full((1, 8))],
        out_specs=_rows(1),
        out_shape=jax.ShapeDtypeStruct((N, 1), jnp.float32),
    )(h, m, p['pool%d_n_g' % d].reshape(1, -1), p['pool%d_n_b' % d].reshape(1, -1),
      p['pool%d_s1_W' % d], p['pool%d_s1_b' % d].reshape(1, -1), s2w, s2b)


def _make_thresh_body(k):
    def body(s_ref, out_ref):
        s = s_ref[...]
        ku = lax.bitcast_convert_type(s, jnp.uint32)
        msb = jnp.uint32(0x80000000)
        ku = jnp.where(ku >= msb, ~ku, ku | msb)
        ks = lax.bitcast_convert_type(ku ^ msb, jnp.int32)

        def cnt_ge(t):
            return jnp.sum((ks >= t).astype(jnp.int32))

        t0 = jnp.where(cnt_ge(jnp.int32(0)) >= k, jnp.int32(0),
                       jnp.int32(-2147483647) - 1)

        def step(i, t):
            cand = t + (jnp.int32(1) << (jnp.int32(30) - i))
            return jnp.where(cnt_ge(cand) >= k, cand, t)

        t = lax.fori_loop(0, 31, step, t0)
        gt = ks > t
        eq = ks == t
        need = jnp.int32(k) - jnp.sum(gt.astype(jnp.int32))
        idx = (lax.broadcasted_iota(jnp.int32, s.shape, 0) * s.shape[1]
               + lax.broadcasted_iota(jnp.int32, s.shape, 1))

        def jstep(i, j):
            cand = j + (jnp.int32(1) << (jnp.int32(13) - i))
            c = jnp.sum((eq & (idx <= cand)).astype(jnp.int32))
            return jnp.where(c <= need, cand, j)

        j = lax.fori_loop(0, 14, jstep, jnp.int32(-1))
        sel = gt | (eq & (idx <= j))
        out_ref[...] = sel.astype(jnp.float32)
    return body


def _topk_mask(s, k):
    spad = jnp.concatenate(
        [s[:, 0], jnp.full((10240 - N,), -1e30, jnp.float32)]).reshape(80, 128)
    m = pl.pallas_call(
        _make_thresh_body(k),
        out_shape=jax.ShapeDtypeStruct((80, 128), jnp.float32),
    )(spad)
    return m.reshape(10240, 1)[:N]


def _final_body(h, g, b, ow, ob, out):
    out[...] = _mm(_ln(h[...], g[...], b[...]), ow[...]) + ob[...]


def _final(h, p):
    return pl.pallas_call(
        _final_body,
        grid=(GRID,),
        in_specs=[_rows(), _full((1, HIDDEN)), _full((1, HIDDEN)),
                  _full((IN_DIM, HIDDEN)), _full((1, IN_DIM))],
        out_specs=_rows(IN_DIM),
        out_shape=jax.ShapeDtypeStruct((N, IN_DIM), jnp.float32),
    )(h, p['out_norm_g'].reshape(1, -1), p['out_norm_b'].reshape(1, -1),
      p['out_proj_W'], p['out_proj_b'].reshape(1, -1))


def kernel(x, edge_index, cond, pos, params):
    p = params
    loops = jnp.arange(N, dtype=jnp.int32)
    src = jnp.concatenate([edge_index[0].astype(jnp.int32), loops])
    dst = jnp.concatenate([edge_index[1].astype(jnp.int32), loops])
    pad = NE_PAD - NE
    srcp = jnp.concatenate([src, jnp.zeros((pad,), jnp.int32)])
    dstp = jnp.concatenate([dst, jnp.full((pad,), jnp.int32(N))])
    trash = STRIDE + jnp.arange(NE_PAD, dtype=jnp.int32) % (ACC_ROWS - STRIDE)
    dmaps = []
    smaps = []
    for q in range(NPASS):
        lo = q * STRIDE
        rel = dstp - lo
        inq = (rel >= 0) & (rel < STRIDE)
        dq = jnp.where(inq, rel, trash)
        sq = jnp.where(inq, srcp, 0)
        dmaps.append(dq.reshape(NTILES, CH_PER_TILE, CHUNK))
        smaps.append(sq.reshape(NTILES, CH_PER_TILE, CHUNK))
    srcs = jnp.stack(smaps)
    dsts = jnp.stack(dmaps)

    def seg_sum(z):
        o = _sc_spmm(HIDDEN, z, srcs, dsts)
        parts = [o[q, :min(STRIDE, N - q * STRIDE)] for q in range(NPASS)]
        return jnp.concatenate(parts, axis=0)

    ones128 = jnp.ones((N, HIDDEN), jnp.float32)
    deg = seg_sum(ones128)[:, 0:1]

    cond8 = jnp.pad(cond, ((0, 7), (0, 0)))
    gammas, betas, cps, cv = _prelude(cond8, p)

    pospad = jnp.pad(pos, ((0, 0), (0, HIDDEN - POS_DIM)))
    h = _input_proj(x, pospad, p, cv)

    def blk(i):
        return (p['blk%d_n1_g' % i].reshape(1, -1), p['blk%d_n1_b' % i].reshape(1, -1),
                p['blk%d_conv_W' % i], p['blk%d_conv_b' % i].reshape(1, -1),
                p['blk%d_n2_g' % i].reshape(1, -1), p['blk%d_n2_b' % i].reshape(1, -1),
                gammas[i:i + 1], betas[i:i + 1], cps[i:i + 1])

    masks = [jnp.ones((N, 1), jnp.float32)]
    skips = []
    off = 0
    n = N
    for d in range(DEPTH):
        n1g, n1b, cw, cb, n2g, n2b, gam, bet, cp = blk(off)
        zs = _pre(h, deg, masks[d], n1g, n1b, cp)
        acc = seg_sum(zs)
        h = _post(h, acc, deg, masks[d], cw, cb, n2g, n2b, gam, bet)
        off += 1
        skips.append(h)
        k = max(1, int(math.ceil(0.5 * n)))
        n = k
        s = _score(h, masks[d], p, d)
        masks.append(_topk_mask(s, k))

    n1g, n1b, cw, cb, n2g, n2b, gam, bet, cp = blk(off)
    zs = _pre(h, deg, masks[DEPTH], n1g, n1b, cp)
    acc = seg_sum(zs)
    h = _post(h, acc, deg, masks[DEPTH], cw, cb, n2g, n2b, gam, bet)
    off += 1

    for d in reversed(range(DEPTH)):
        n1g, n1b, cw, cb, n2g, n2b, gam, bet, cp = blk(off)
        heff, zs = _pre_unpool(h, skips[d], masks[d + 1], deg, masks[d],
                               n1g, n1b, cp)
        acc = seg_sum(zs)
        h = _post(heff, acc, deg, masks[d], cw, cb, n2g, n2b, gam, bet)
        off += 1

    return _final(h, p)


# speedup vs baseline: 38.0009x; 38.0009x over previous
import functools
import math

import jax
import jax.numpy as jnp
from jax import lax
from jax.experimental import pallas as pl
from jax.experimental.pallas import tpu as pltpu
from jax.experimental.pallas import tpu_sc as plsc

N = 10000
E = 320000
IN_DIM = 128
HIDDEN = 128
DEPTH = 3
TOTAL_BLOCKS = 7
POS_DIM = 3
WIDTH = 512

NE = E + N
CHUNK = 64
NTILES = 16
NE_PAD = ((NE + NTILES * CHUNK - 1) // (NTILES * CHUNK)) * (NTILES * CHUNK)
CH_PER_TILE = NE_PAD // (NTILES * CHUNK)
NPASS = 2
STRIDE = 5008
ACC_ROWS = 5120
RPT = ACC_ROWS // 16
TRASH = ACC_ROWS - 1

BR = 2000
GRID = N // BR


def _make_sc_spmm(d):
    mesh = plsc.VectorSubcoreMesh(core_axis_name="c", subcore_axis_name="s")

    @functools.partial(
        pl.kernel,
        out_type=jax.ShapeDtypeStruct((NPASS, ACC_ROWS, d), jnp.float32),
        mesh=mesh,
        scratch_types=[
            pltpu.VMEM((3, CHUNK), jnp.int32),
            pltpu.VMEM((3, CHUNK), jnp.int32),
            pltpu.VMEM((2, CHUNK, d), jnp.float32),
            pltpu.VMEM_SHARED((ACC_ROWS, d), jnp.float32),
            pltpu.SemaphoreType.DMA,
            pltpu.SemaphoreType.DMA,
        ],
    )
    def spmm(z_hbm, srcs_hbm, dsts_hbm, out_hbm,
             sidx, didx, rbuf, acc, isem, gsem):
        wid = lax.axis_index("s")
        p = lax.axis_index("c")
        r0 = wid * RPT
        zv = jnp.zeros((16,), jnp.float32)

        def idx_start(p, j, slot):
            pltpu.async_copy(srcs_hbm.at[wid].at[j], sidx.at[slot], isem)
            pltpu.async_copy(dsts_hbm.at[p].at[wid].at[j], didx.at[slot], isem)

        def idx_wait(p, j, slot):
            pltpu.make_async_copy(srcs_hbm.at[wid].at[j], sidx.at[slot],
                                  isem).wait()
            pltpu.make_async_copy(dsts_hbm.at[p].at[wid].at[j], didx.at[slot],
                                  isem).wait()

        if True:
            @pl.loop(0, CHUNK)
            def _(r):
                @pl.loop(0, d // 16)
                def _(cc):
                    rbuf.at[0].at[r][pl.ds(cc * 16, 16)] = zv

            @pl.loop(0, RPT // CHUNK)
            def _(i):
                pltpu.sync_copy(rbuf.at[0],
                                acc.at[pl.ds(r0 + i * CHUNK, CHUNK)])
            plsc.subcore_barrier()

            idx_start(p, 0, 0)
            idx_start(p, 1, 1)
            idx_wait(p, 0, 0)
            pltpu.async_copy(z_hbm.at[sidx.at[0]], rbuf.at[0], gsem)

            @pl.loop(0, CH_PER_TILE)
            def _(j):
                b = lax.rem(j, 2)
                nb = 1 - b
                s2 = lax.rem(j + 2, 3)
                s1 = lax.rem(j + 1, 3)

                @pl.when(j + 2 < CH_PER_TILE)
                def _():
                    idx_start(p, j + 2, s2)

                @pl.when(j + 1 < CH_PER_TILE)
                def _():
                    idx_wait(p, j + 1, s1)
                    pltpu.async_copy(z_hbm.at[sidx.at[s1]], rbuf.at[nb], gsem)

                pltpu.make_async_copy(z_hbm.at[sidx.at[lax.rem(j, 3)]],
                                      rbuf.at[b], gsem).wait()
                pltpu.sync_copy(rbuf.at[b], acc.at[didx.at[lax.rem(j, 3)]],
                                add=True)

            plsc.subcore_barrier()
            pltpu.sync_copy(acc.at[pl.ds(r0, RPT)],
                            out_hbm.at[p].at[pl.ds(r0, RPT)])
            plsc.subcore_barrier()

    return spmm


_SC_CACHE = {}


def _sc_spmm(d, *args):
    if d not in _SC_CACHE:
        _SC_CACHE[d] = _make_sc_spmm(d)
    return _SC_CACHE[d](*args)


def _mm(x, w):
    return lax.dot_general(x, w, (((1,), (1,)), ((), ())),
                           preferred_element_type=jnp.float32)


def _ln(z, g, b, eps=1e-5):
    m = jnp.mean(z, axis=-1, keepdims=True)
    v = jnp.mean((z - m) ** 2, axis=-1, keepdims=True)
    return (z - m) / jnp.sqrt(v + eps) * g + b


def _silu(x):
    return x * (1.0 / (1.0 + jnp.exp(-x)))


def _scale_of(deg, m):
    return m * lax.rsqrt(jnp.maximum(deg, 1.0))


def _full(shape):
    return pl.BlockSpec(shape, lambda i: (0, 0))


def _rows(width=HIDDEN):
    return pl.BlockSpec((BR, width), lambda i: (i, 0))


def _prelude_body(cond_ref, f1w, f1b, f2w, f2b, f3w, f3b,
                  c1w, c1b, c2w, c2b, cpw, cpb,
                  f_out, cv_out, cp_out):
    c8 = cond_ref[...]
    t = _silu(_mm(c8, f1w[...]) + f1b[...])
    t = _silu(_mm(t, f2w[...]) + f2b[...])
    f_out[...] = _mm(t, f3w[...]) + f3b[...]
    cv_out[...] = _mm(_silu(_mm(c8, c1w[...]) + c1b[...]), c2w[...]) + c2b[...]
    cp_out[...] = _mm(c8, cpw[...]) + cpb[...]


def _prelude(cond8, p):
    cpw = jnp.concatenate([p['blk%d_cp_W' % i] for i in range(TOTAL_BLOCKS)], 0)
    cpb = jnp.concatenate([p['blk%d_cp_b' % i] for i in range(TOTAL_BLOCKS)], 0)
    nf = TOTAL_BLOCKS * 2 * HIDDEN
    f, cv, cp = pl.pallas_call(
        _prelude_body,
        out_shape=[jax.ShapeDtypeStruct((8, nf), jnp.float32),
                   jax.ShapeDtypeStruct((8, HIDDEN), jnp.float32),
                   jax.ShapeDtypeStruct((8, TOTAL_BLOCKS * HIDDEN), jnp.float32)],
    )(cond8,
      p['film1_W'], p['film1_b'].reshape(1, -1),
      p['film2_W'], p['film2_b'].reshape(1, -1),
      p['film3_W'], p['film3_b'].reshape(1, -1),
      p['cond1_W'], p['cond1_b'].reshape(1, -1),
      p['cond2_W'], p['cond2_b'].reshape(1, -1),
      cpw, cpb.reshape(1, -1))
    f = f[0].reshape(TOTAL_BLOCKS, 2, HIDDEN)
    gammas = 1.0 + f[:, 0, :]
    betas = f[:, 1, :]
    cps = cp[0].reshape(TOTAL_BLOCKS, HIDDEN)
    return gammas, betas, cps, cv[0:1]


def _input_body(x, pos, inw, inb, p1w, p1b, p2w, p2b, cv, out):
    h = _mm(x[...], inw[...]) + inb[...]
    h = h + _mm(_silu(_mm(pos[...], p1w[...]) + p1b[...]), p2w[...]) + p2b[...]
    out[...] = h + cv[...]


def _input_proj(x, pospad, p, cv):
    p1w = jnp.pad(p['pos1_W'], ((0, 0), (0, HIDDEN - POS_DIM)))
    return pl.pallas_call(
        _input_body,
        grid=(GRID,),
        in_specs=[_rows(), _rows(), _full((HIDDEN, IN_DIM)), _full((1, HIDDEN)),
                  _full((HIDDEN, HIDDEN)), _full((1, HIDDEN)),
                  _full((HIDDEN, HIDDEN)), _full((1, HIDDEN)),
                  _full((1, HIDDEN))],
        out_specs=_rows(),
        out_shape=jax.ShapeDtypeStruct((N, HIDDEN), jnp.float32),
    )(x, pospad, p['in_proj_W'], p['in_proj_b'].reshape(1, -1),
      p1w, p['pos1_b'].reshape(1, -1),
      p['pos2_W'], p['pos2_b'].reshape(1, -1), cv)


def _pre_body(h, deg, m, g, b, cp, zs):
    z = _silu(_ln(h[...], g[...], b[...]) + cp[...])
    zs[...] = z * _scale_of(deg[...], m[...])


def _pre(h, deg, m, g, b, cp):
    return pl.pallas_call(
        _pre_body,
        grid=(GRID,),
        in_specs=[_rows(), _rows(1), _rows(1), _full((1, HIDDEN)),
                  _full((1, HIDDEN)), _full((1, HIDDEN))],
        out_specs=_rows(),
        out_shape=jax.ShapeDtypeStruct((N, HIDDEN), jnp.float32),
    )(h, deg, m, g, b, cp)


def _preu_body(h, skip, mn, deg, m, g, b, cp, heff_o, zs):
    heff = jnp.where(mn[...] > 0, h[...], 0.0) + skip[...]
    heff_o[...] = heff
    z = _silu(_ln(heff, g[...], b[...]) + cp[...])
    zs[...] = z * _scale_of(deg[...], m[...])


def _pre_unpool(h, skip, mn, deg, m, g, b, cp):
    return pl.pallas_call(
        _preu_body,
        grid=(GRID,),
        in_specs=[_rows(), _rows(), _rows(1), _rows(1), _rows(1),
                  _full((1, HIDDEN)), _full((1, HIDDEN)), _full((1, HIDDEN))],
        out_specs=[_rows(), _rows()],
        out_shape=[jax.ShapeDtypeStruct((N, HIDDEN), jnp.float32),
                   jax.ShapeDtypeStruct((N, HIDDEN), jnp.float32)],
    )(h, skip, mn, deg, m, g, b, cp)


def _post_body(h, a0, deg, m, cw, cb, g, b, gam, bet, out):
    gsum = a0[...] * _scale_of(deg[...], m[...])
    z = _mm(gsum, cw[...]) + cb[...]
    z = _ln(z, g[...], b[...])
    out[...] = h[...] + z * gam[...] + bet[...]


def _post(h, acc, deg, m, cw, cb, g, b, gam, bet):
    return pl.pallas_call(
        _post_body,
        grid=(GRID,),
        in_specs=[_rows(), _rows(), _rows(1), _rows(1),
                  _full((HIDDEN, HIDDEN)), _full((1, HIDDEN)),
                  _full((1, HIDDEN)), _full((1, HIDDEN)),
                  _full((1, HIDDEN)), _full((1, HIDDEN))],
        out_specs=_rows(),
        out_shape=jax.ShapeDtypeStruct((N, HIDDEN), jnp.float32),
    )(h, acc, deg, m, cw, cb, g, b, gam, bet)


def _score_body(h, m, g, b, s1w, s1b, s2w, s2b, out):
    z = _ln(h[...], g[...], b[...])
    z = _silu(_mm(z, s1w[...]) + s1b[...])
    s = _mm(z, s2w[...]) + s2b[...]
    out[...] = jnp.where(m[...] > 0, s[:, 0:1], -1e30)


def _score(h, m, p, d):
    s2w = jnp.pad(p['pool%d_s2_W' % d], ((0, 7), (0, 0)))
    s2b = jnp.broadcast_to(p['pool%d_s2_b' % d].reshape(1, 1), (1, 8))
    return pl.pallas_call(
        _score_body,
        grid=(GRID,),
        in_specs=[_rows(), _rows(1), _full((1, HIDDEN)), _full((1, HIDDEN)),
                  _full((HIDDEN // 2, HIDDEN)), _full((1, HIDDEN // 2)),
                  _full((8, HIDDEN // 2)), _full((1, 8))],
        out_specs=_rows(1),
        out_shape=jax.ShapeDtypeStruct((N, 1), jnp.float32),
    )(h, m, p['pool%d_n_g' % d].reshape(1, -1), p['pool%d_n_b' % d].reshape(1, -1),
      p['pool%d_s1_W' % d], p['pool%d_s1_b' % d].reshape(1, -1), s2w, s2b)


def _make_thresh_body(k):
    def body(s_ref, out_ref):
        s = s_ref[...]
        ku = lax.bitcast_convert_type(s, jnp.uint32)
        msb = jnp.uint32(0x80000000)
        ku = jnp.where(ku >= msb, ~ku, ku | msb)
        ks = lax.bitcast_convert_type(ku ^ msb, jnp.int32)

        def cnt_ge(t):
            return jnp.sum((ks >= t).astype(jnp.int32))

        t0 = jnp.where(cnt_ge(jnp.int32(0)) >= k, jnp.int32(0),
                       jnp.int32(-2147483647) - 1)

        def step(i, t):
            cand = t + (jnp.int32(1) << (jnp.int32(30) - i))
            return jnp.where(cnt_ge(cand) >= k, cand, t)

        t = lax.fori_loop(0, 31, step, t0)
        gt = ks > t
        eq = ks == t
        need = jnp.int32(k) - jnp.sum(gt.astype(jnp.int32))
        idx = (lax.broadcasted_iota(jnp.int32, s.shape, 0) * s.shape[1]
               + lax.broadcasted_iota(jnp.int32, s.shape, 1))

        def jstep(i, j):
            cand = j + (jnp.int32(1) << (jnp.int32(13) - i))
            c = jnp.sum((eq & (idx <= cand)).astype(jnp.int32))
            return jnp.where(c <= need, cand, j)

        j = lax.fori_loop(0, 14, jstep, jnp.int32(-1))
        sel = gt | (eq & (idx <= j))
        out_ref[...] = sel.astype(jnp.float32)
    return body


def _topk_mask(s, k):
    spad = jnp.concatenate(
        [s[:, 0], jnp.full((10240 - N,), -1e30, jnp.float32)]).reshape(80, 128)
    m = pl.pallas_call(
        _make_thresh_body(k),
        out_shape=jax.ShapeDtypeStruct((80, 128), jnp.float32),
    )(spad)
    return m.reshape(10240, 1)[:N]


def _final_body(h, g, b, ow, ob, out):
    out[...] = _mm(_ln(h[...], g[...], b[...]), ow[...]) + ob[...]


def _final(h, p):
    return pl.pallas_call(
        _final_body,
        grid=(GRID,),
        in_specs=[_rows(), _full((1, HIDDEN)), _full((1, HIDDEN)),
                  _full((IN_DIM, HIDDEN)), _full((1, IN_DIM))],
        out_specs=_rows(IN_DIM),
        out_shape=jax.ShapeDtypeStruct((N, IN_DIM), jnp.float32),
    )(h, p['out_norm_g'].reshape(1, -1), p['out_norm_b'].reshape(1, -1),
      p['out_proj_W'], p['out_proj_b'].reshape(1, -1))


def kernel(x, edge_index, cond, pos, params):
    p = params
    loops = jnp.arange(N, dtype=jnp.int32)
    src = jnp.concatenate([edge_index[0].astype(jnp.int32), loops])
    dst = jnp.concatenate([edge_index[1].astype(jnp.int32), loops])
    pad = NE_PAD - NE
    srcs = jnp.concatenate([src, jnp.zeros((pad,), jnp.int32)])
    dstp = jnp.concatenate([dst, jnp.full((pad,), jnp.int32(N))])
    trash = STRIDE + jnp.arange(NE_PAD, dtype=jnp.int32) % (ACC_ROWS - STRIDE)
    dmaps = []
    for q in range(NPASS):
        lo = q * STRIDE
        rel = dstp - lo
        dq = jnp.where((rel >= 0) & (rel < STRIDE), rel, trash)
        dmaps.append(dq.reshape(NTILES, CH_PER_TILE, CHUNK))
    srcs = srcs.reshape(NTILES, CH_PER_TILE, CHUNK)
    dsts = jnp.stack(dmaps)

    def seg_sum(z):
        o = _sc_spmm(HIDDEN, z, srcs, dsts)
        parts = [o[q, :min(STRIDE, N - q * STRIDE)] for q in range(NPASS)]
        return jnp.concatenate(parts, axis=0)

    ones128 = jnp.ones((N, HIDDEN), jnp.float32)
    deg = seg_sum(ones128)[:, 0:1]

    cond8 = jnp.pad(cond, ((0, 7), (0, 0)))
    gammas, betas, cps, cv = _prelude(cond8, p)

    pospad = jnp.pad(pos, ((0, 0), (0, HIDDEN - POS_DIM)))
    h = _input_proj(x, pospad, p, cv)

    def blk(i):
        return (p['blk%d_n1_g' % i].reshape(1, -1), p['blk%d_n1_b' % i].reshape(1, -1),
                p['blk%d_conv_W' % i], p['blk%d_conv_b' % i].reshape(1, -1),
                p['blk%d_n2_g' % i].reshape(1, -1), p['blk%d_n2_b' % i].reshape(1, -1),
                gammas[i:i + 1], betas[i:i + 1], cps[i:i + 1])

    masks = [jnp.ones((N, 1), jnp.float32)]
    skips = []
    off = 0
    n = N
    for d in range(DEPTH):
        n1g, n1b, cw, cb, n2g, n2b, gam, bet, cp = blk(off)
        zs = _pre(h, deg, masks[d], n1g, n1b, cp)
        acc = seg_sum(zs)
        h = _post(h, acc, deg, masks[d], cw, cb, n2g, n2b, gam, bet)
        off += 1
        skips.append(h)
        k = max(1, int(math.ceil(0.5 * n)))
        n = k
        s = _score(h, masks[d], p, d)
        masks.append(_topk_mask(s, k))

    n1g, n1b, cw, cb, n2g, n2b, gam, bet, cp = blk(off)
    zs = _pre(h, deg, masks[DEPTH], n1g, n1b, cp)
    acc = seg_sum(zs)
    h = _post(h, acc, deg, masks[DEPTH], cw, cb, n2g, n2b, gam, bet)
    off += 1

    for d in reversed(range(DEPTH)):
        n1g, n1b, cw, cb, n2g, n2b, gam, bet, cp = blk(off)
        heff, zs = _pre_unpool(h, skips[d], masks[d + 1], deg, masks[d],
                               n1g, n1b, cp)
        acc = seg_sum(zs)
        h = _post(heff, acc, deg, masks[d], cw, cb, n2g, n2b, gam, bet)
        off += 1

    return _final(h, p)


# speedup vs baseline: 38.3223x; 1.0085x over previous
import dataclasses
import functools
import math

import jax
import jax.numpy as jnp
from jax import lax
from jax.experimental import pallas as pl
from jax.experimental.pallas import tpu as pltpu
from jax.experimental.pallas import tpu_sc as plsc

N = 10000
E = 320000
IN_DIM = 128
HIDDEN = 128
DEPTH = 3
TOTAL_BLOCKS = 7
POS_DIM = 3
WIDTH = 512

NE = E + N
CHUNK = 64
NTILES = 16
NE_PAD = ((NE + 32 * CHUNK - 1) // (32 * CHUNK)) * (32 * CHUNK)
CH_PER_TILE = NE_PAD // (NTILES * CHUNK)
STRIDE = 5008
ACC_ROWS = 5120
RPT = ACC_ROWS // 16
SEG = NE_PAD // 32
SEGP = ((SEG + CHUNK - 1) // CHUNK) * CHUNK
ICH = 32

BR = 2000
GRID = N // BR


def _sc_mesh():
    return plsc.VectorSubcoreMesh(core_axis_name="c", subcore_axis_name="s")


def _make_sc_partition():
    cp = pltpu.CompilerParams()
    if "needs_layout_passes" in pltpu.CompilerParams.__dataclass_fields__:
        cp = dataclasses.replace(cp, needs_layout_passes=False)

    @functools.partial(
        pl.kernel,
        out_type=[jax.ShapeDtypeStruct((2, 32, SEGP), jnp.int32),
                  jax.ShapeDtypeStruct((2, 32, SEGP), jnp.int32),
                  jax.ShapeDtypeStruct((2, 32, 16), jnp.int32)],
        mesh=_sc_mesh(),
        compiler_params=cp,
        scratch_types=[
            pltpu.VMEM((2, ICH), jnp.int32),
            pltpu.VMEM((2, ICH), jnp.int32),
            pltpu.VMEM((SEGP,), jnp.int32),
            pltpu.VMEM((SEGP,), jnp.int32),
            pltpu.VMEM((SEGP,), jnp.int32),
            pltpu.VMEM((SEGP,), jnp.int32),
            pltpu.VMEM((16,), jnp.int32),
            pltpu.SemaphoreType.DMA,
        ],
    )
    def part(src_hbm, dst_hbm, src_out, dst_out, cnt_out,
             sring, dring, slo, dlo, shi, dhi, cbuf, isem):
        c = lax.axis_index("c")
        t = lax.axis_index("s")
        sub = c * NTILES + t
        base = sub * SEG
        iota = lax.broadcasted_iota(jnp.int32, (16,), 0)
        zeros16 = jnp.zeros((16,), jnp.int32)
        ntrash = ACC_ROWS - STRIDE

        @pl.loop(0, SEGP // 16)
        def _(i):
            tr = STRIDE + lax.rem(i * 16 + iota, ntrash)
            slo[pl.ds(i * 16, 16)] = zeros16
            shi[pl.ds(i * 16, 16)] = zeros16
            dlo[pl.ds(i * 16, 16)] = tr
            dhi[pl.ds(i * 16, 16)] = tr

        def in_start(j, slot):
            pltpu.async_copy(src_hbm.at[pl.ds(base + j * ICH, ICH)],
                             sring.at[slot], isem)
            pltpu.async_copy(dst_hbm.at[pl.ds(base + j * ICH, ICH)],
                             dring.at[slot], isem)

        def in_wait(j, slot):
            pltpu.make_async_copy(src_hbm.at[pl.ds(base + j * ICH, ICH)],
                                  sring.at[slot], isem).wait()
            pltpu.make_async_copy(dst_hbm.at[pl.ds(base + j * ICH, ICH)],
                                  dring.at[slot], isem).wait()

        in_start(0, 0)

        def chunk_body(j, cur):
            cl, ch = cur
            b = lax.rem(j, 2)

            @pl.when(j + 1 < SEG // ICH)
            def _():
                in_start(j + 1, 1 - b)

            in_wait(j, b)
            for g in range(ICH // 16):
                s16 = sring.at[b][pl.ds(g * 16, 16)]
                d16 = dring.at[b][pl.ds(g * 16, 16)]
                m = d16 < STRIDE
                nlo = jnp.max(plsc.all_reduce_population_count(m))
                plsc.store_compressed(slo.at[pl.ds(cl, 16)], s16, mask=m)
                plsc.store_compressed(dlo.at[pl.ds(cl, 16)], d16, mask=m)
                plsc.store_compressed(shi.at[pl.ds(ch, 16)], s16, mask=~m)
                plsc.store_compressed(dhi.at[pl.ds(ch, 16)], d16 - STRIDE, mask=~m)
                cl = cl + nlo
                ch = ch + (16 - nlo)
            return (cl, ch)

        cl, ch = lax.fori_loop(0, SEG // ICH, chunk_body,
                               (jnp.int32(0), jnp.int32(0)))

        pltpu.sync_copy(slo, src_out.at[0].at[sub])
        pltpu.sync_copy(dlo, dst_out.at[0].at[sub])
        pltpu.sync_copy(shi, src_out.at[1].at[sub])
        pltpu.sync_copy(dhi, dst_out.at[1].at[sub])
        cbuf[pl.ds(0, 16)] = zeros16 + cl
        pltpu.sync_copy(cbuf, cnt_out.at[0].at[sub])
        cbuf[pl.ds(0, 16)] = zeros16 + ch
        pltpu.sync_copy(cbuf, cnt_out.at[1].at[sub])

    return part


def _make_sc_spmm(d):
    @functools.partial(
        pl.kernel,
        out_type=jax.ShapeDtypeStruct((2, ACC_ROWS, d), jnp.float32),
        mesh=_sc_mesh(),
        scratch_types=[
            pltpu.VMEM((3, CHUNK), jnp.int32),
            pltpu.VMEM((3, CHUNK), jnp.int32),
            pltpu.VMEM((2, CHUNK, d), jnp.float32),
            pltpu.VMEM((16,), jnp.int32),
            pltpu.VMEM_SHARED((ACC_ROWS, d), jnp.float32),
            pltpu.SemaphoreType.DMA,
            pltpu.SemaphoreType.DMA,
        ],
    )
    def spmm(z_hbm, srcs_hbm, dsts_hbm, cnts_hbm, out_hbm,
             sidx, didx, rbuf, cbuf, acc, isem, gsem):
        wid = lax.axis_index("s")
        p = lax.axis_index("c")
        r0 = wid * RPT
        zv = jnp.zeros((16,), jnp.float32)

        @pl.loop(0, CHUNK)
        def _(r):
            @pl.loop(0, d // 16)
            def _(cc):
                rbuf.at[0].at[r][pl.ds(cc * 16, 16)] = zv

        @pl.loop(0, RPT // CHUNK)
        def _(i):
            pltpu.sync_copy(rbuf.at[0], acc.at[pl.ds(r0 + i * CHUNK, CHUNK)])
        plsc.subcore_barrier()

        for si in range(2):
            sub = 2 * wid + si
            pltpu.sync_copy(cnts_hbm.at[p].at[sub], cbuf)
            cnt = cbuf[pl.ds(0, 16)][0]
            nch = (cnt + CHUNK - 1) // CHUNK
            srcl = srcs_hbm.at[p].at[sub]
            dstl = dsts_hbm.at[p].at[sub]

            def idx_start(j, slot):
                pltpu.async_copy(srcl.at[pl.ds(j * CHUNK, CHUNK)],
                                 sidx.at[slot], isem)
                pltpu.async_copy(dstl.at[pl.ds(j * CHUNK, CHUNK)],
                                 didx.at[slot], isem)

            def idx_wait(j, slot):
                pltpu.make_async_copy(srcl.at[pl.ds(j * CHUNK, CHUNK)],
                                      sidx.at[slot], isem).wait()
                pltpu.make_async_copy(dstl.at[pl.ds(j * CHUNK, CHUNK)],
                                      didx.at[slot], isem).wait()

            @pl.when(nch >= 1)
            def _():
                idx_start(0, 0)

                @pl.when(nch >= 2)
                def _():
                    idx_start(1, 1)
                idx_wait(0, 0)
                pltpu.async_copy(z_hbm.at[sidx.at[0]], rbuf.at[0], gsem)

                @pl.loop(0, nch)
                def _(j):
                    b = lax.rem(j, 2)
                    nb = 1 - b
                    s2 = lax.rem(j + 2, 3)
                    s1 = lax.rem(j + 1, 3)

                    @pl.when(j + 2 < nch)
                    def _():
                        idx_start(j + 2, s2)

                    @pl.when(j + 1 < nch)
                    def _():
                        idx_wait(j + 1, s1)
                        pltpu.async_copy(z_hbm.at[sidx.at[s1]], rbuf.at[nb],
                                         gsem)

                    pltpu.make_async_copy(z_hbm.at[sidx.at[lax.rem(j, 3)]],
                                          rbuf.at[b], gsem).wait()
                    pltpu.sync_copy(rbuf.at[b], acc.at[didx.at[lax.rem(j, 3)]],
                                    add=True)

        plsc.subcore_barrier()
        pltpu.sync_copy(acc.at[pl.ds(r0, RPT)],
                        out_hbm.at[p].at[pl.ds(r0, RPT)])

    return spmm


_SC_CACHE = {}


def _sc_spmm(d, *args):
    if d not in _SC_CACHE:
        _SC_CACHE[d] = _make_sc_spmm(d)
    return _SC_CACHE[d](*args)


def _mm(x, w):
    return lax.dot_general(x, w, (((1,), (1,)), ((), ())),
                           preferred_element_type=jnp.float32)


def _ln(z, g, b, eps=1e-5):
    m = jnp.mean(z, axis=-1, keepdims=True)
    v = jnp.mean((z - m) ** 2, axis=-1, keepdims=True)
    return (z - m) / jnp.sqrt(v + eps) * g + b


def _silu(x):
    return x * (1.0 / (1.0 + jnp.exp(-x)))


def _scale_of(deg, m):
    return m * lax.rsqrt(jnp.maximum(deg, 1.0))


def _full(shape):
    return pl.BlockSpec(shape, lambda i: (0, 0))


def _rows(width=HIDDEN):
    return pl.BlockSpec((BR, width), lambda i: (i, 0))


def _prelude_body(cond_ref, f1w, f1b, f2w, f2b, f3w, f3b,
                  c1w, c1b, c2w, c2b, cpw, cpb,
                  f_out, cv_out, cp_out):
    c8 = cond_ref[...]
    t = _silu(_mm(c8, f1w[...]) + f1b[...])
    t = _silu(_mm(t, f2w[...]) + f2b[...])
    f_out[...] = _mm(t, f3w[...]) + f3b[...]
    cv_out[...] = _mm(_silu(_mm(c8, c1w[...]) + c1b[...]), c2w[...]) + c2b[...]
    cp_out[...] = _mm(c8, cpw[...]) + cpb[...]


def _prelude(cond8, p):
    cpw = jnp.concatenate([p['blk%d_cp_W' % i] for i in range(TOTAL_BLOCKS)], 0)
    cpb = jnp.concatenate([p['blk%d_cp_b' % i] for i in range(TOTAL_BLOCKS)], 0)
    nf = TOTAL_BLOCKS * 2 * HIDDEN
    f, cv, cp = pl.pallas_call(
        _prelude_body,
        out_shape=[jax.ShapeDtypeStruct((8, nf), jnp.float32),
                   jax.ShapeDtypeStruct((8, HIDDEN), jnp.float32),
                   jax.ShapeDtypeStruct((8, TOTAL_BLOCKS * HIDDEN), jnp.float32)],
    )(cond8,
      p['film1_W'], p['film1_b'].reshape(1, -1),
      p['film2_W'], p['film2_b'].reshape(1, -1),
      p['film3_W'], p['film3_b'].reshape(1, -1),
      p['cond1_W'], p['cond1_b'].reshape(1, -1),
      p['cond2_W'], p['cond2_b'].reshape(1, -1),
      cpw, cpb.reshape(1, -1))
    f = f[0].reshape(TOTAL_BLOCKS, 2, HIDDEN)
    gammas = 1.0 + f[:, 0, :]
    betas = f[:, 1, :]
    cps = cp[0].reshape(TOTAL_BLOCKS, HIDDEN)
    return gammas, betas, cps, cv[0:1]


def _input_body(x, pos, inw, inb, p1w, p1b, p2w, p2b, cv, out):
    h = _mm(x[...], inw[...]) + inb[...]
    h = h + _mm(_silu(_mm(pos[...], p1w[...]) + p1b[...]), p2w[...]) + p2b[...]
    out[...] = h + cv[...]


def _input_proj(x, pospad, p, cv):
    p1w = jnp.pad(p['pos1_W'], ((0, 0), (0, HIDDEN - POS_DIM)))
    return pl.pallas_call(
        _input_body,
        grid=(GRID,),
        in_specs=[_rows(), _rows(), _full((HIDDEN, IN_DIM)), _full((1, HIDDEN)),
                  _full((HIDDEN, HIDDEN)), _full((1, HIDDEN)),
                  _full((HIDDEN, HIDDEN)), _full((1, HIDDEN)),
                  _full((1, HIDDEN))],
        out_specs=_rows(),
        out_shape=jax.ShapeDtypeStruct((N, HIDDEN), jnp.float32),
    )(x, pospad, p['in_proj_W'], p['in_proj_b'].reshape(1, -1),
      p1w, p['pos1_b'].reshape(1, -1),
      p['pos2_W'], p['pos2_b'].reshape(1, -1), cv)


def _pre_body(h, deg, m, g, b, cp, zs):
    z = _silu(_ln(h[...], g[...], b[...]) + cp[...])
    zs[...] = z * _scale_of(deg[...], m[...])


def _pre(h, deg, m, g, b, cp):
    return pl.pallas_call(
        _pre_body,
        grid=(GRID,),
        in_specs=[_rows(), _rows(1), _rows(1), _full((1, HIDDEN)),
                  _full((1, HIDDEN)), _full((1, HIDDEN))],
        out_specs=_rows(),
        out_shape=jax.ShapeDtypeStruct((N, HIDDEN), jnp.float32),
    )(h, deg, m, g, b, cp)


def _preu_body(h, skip, mn, deg, m, g, b, cp, heff_o, zs):
    heff = jnp.where(mn[...] > 0, h[...], 0.0) + skip[...]
    heff_o[...] = heff
    z = _silu(_ln(heff, g[...], b[...]) + cp[...])
    zs[...] = z * _scale_of(deg[...], m[...])


def _pre_unpool(h, skip, mn, deg, m, g, b, cp):
    return pl.pallas_call(
        _preu_body,
        grid=(GRID,),
        in_specs=[_rows(), _rows(), _rows(1), _rows(1), _rows(1),
                  _full((1, HIDDEN)), _full((1, HIDDEN)), _full((1, HIDDEN))],
        out_specs=[_rows(), _rows()],
        out_shape=[jax.ShapeDtypeStruct((N, HIDDEN), jnp.float32),
                   jax.ShapeDtypeStruct((N, HIDDEN), jnp.float32)],
    )(h, skip, mn, deg, m, g, b, cp)


def _post_body(h, a0, deg, m, cw, cb, g, b, gam, bet, out):
    gsum = a0[...] * _scale_of(deg[...], m[...])
    z = _mm(gsum, cw[...]) + cb[...]
    z = _ln(z, g[...], b[...])
    out[...] = h[...] + z * gam[...] + bet[...]


def _post(h, acc, deg, m, cw, cb, g, b, gam, bet):
    return pl.pallas_call(
        _post_body,
        grid=(GRID,),
        in_specs=[_rows(), _rows(), _rows(1), _rows(1),
                  _full((HIDDEN, HIDDEN)), _full((1, HIDDEN)),
                  _full((1, HIDDEN)), _full((1, HIDDEN)),
                  _full((1, HIDDEN)), _full((1, HIDDEN))],
        out_specs=_rows(),
        out_shape=jax.ShapeDtypeStruct((N, HIDDEN), jnp.float32),
    )(h, acc, deg, m, cw, cb, g, b, gam, bet)


def _score_body(h, m, g, b, s1w, s1b, s2w, s2b, out):
    z = _ln(h[...], g[...], b[...])
    z = _silu(_mm(z, s1w[...]) + s1b[...])
    s = _mm(z, s2w[...]) + s2b[...]
    out[...] = jnp.where(m[...] > 0, s[:, 0:1], -1e30)


def _score(h, m, p, d):
    s2w = jnp.pad(p['pool%d_s2_W' % d], ((0, 7), (0, 0)))
    s2b = jnp.broadcast_to(p['pool%d_s2_b' % d].reshape(1, 1), (1, 8))
    return pl.pallas_call(
        _score_body,
        grid=(GRID,),
        in_specs=[_rows(), _rows(1), _full((1, HIDDEN)), _full((1, HIDDEN)),
                  _full((HIDDEN // 2, HIDDEN)), _full((1, HIDDEN // 2)),
                  _full((8, HIDDEN // 2)), _full((1, 8))],
        out_specs=_rows(1),
        out_shape=jax.ShapeDtypeStruct((N, 1), jnp.float32),
    )(h, m, p['pool%d_n_g' % d].reshape(1, -1), p['pool%d_n_b' % d].reshape(1, -1),
      p['pool%d_s1_W' % d], p['pool%d_s1_b' % d].reshape(1, -1), s2w, s2b)


def _make_thresh_body(k):
    def body(s_ref, out_ref):
        s = s_ref[...]
        ku = lax.bitcast_convert_type(s, jnp.uint32)
        msb = jnp.uint32(0x80000000)
        ku = jnp.where(ku >= msb, ~ku, ku | msb)
        ks = lax.bitcast_convert_type(ku ^ msb, jnp.int32)

        def cnt_ge(t):
            return jnp.sum((ks >= t).astype(jnp.int32))

        t0 = jnp.where(cnt_ge(jnp.int32(0)) >= k, jnp.int32(0),
                       jnp.int32(-2147483647) - 1)

        def step(i, t):
            cand = t + (jnp.int32(1) << (jnp.int32(30) - i))
            return jnp.where(cnt_ge(cand) >= k, cand, t)

        t = lax.fori_loop(0, 31, step, t0)
        gt = ks > t
        eq = ks == t
        need = jnp.int32(k) - jnp.sum(gt.astype(jnp.int32))
        idx = (lax.broadcasted_iota(jnp.int32, s.shape, 0) * s.shape[1]
               + lax.broadcasted_iota(jnp.int32, s.shape, 1))

        def jstep(i, j):
            cand = j + (jnp.int32(1) << (jnp.int32(13) - i))
            c = jnp.sum((eq & (idx <= cand)).astype(jnp.int32))
            return jnp.where(c <= need, cand, j)

        j = lax.fori_loop(0, 14, jstep, jnp.int32(-1))
        sel = gt | (eq & (idx <= j))
        out_ref[...] = sel.astype(jnp.float32)
    return body


def _topk_mask(s, k):
    spad = jnp.concatenate(
        [s[:, 0], jnp.full((10240 - N,), -1e30, jnp.float32)]).reshape(80, 128)
    m = pl.pallas_call(
        _make_thresh_body(k),
        out_shape=jax.ShapeDtypeStruct((80, 128), jnp.float32),
    )(spad)
    return m.reshape(10240, 1)[:N]


def _final_body(h, g, b, ow, ob, out):
    out[...] = _mm(_ln(h[...], g[...], b[...]), ow[...]) + ob[...]


def _final(h, p):
    return pl.pallas_call(
        _final_body,
        grid=(GRID,),
        in_specs=[_rows(), _full((1, HIDDEN)), _full((1, HIDDEN)),
                  _full((IN_DIM, HIDDEN)), _full((1, IN_DIM))],
        out_specs=_rows(IN_DIM),
        out_shape=jax.ShapeDtypeStruct((N, IN_DIM), jnp.float32),
    )(h, p['out_norm_g'].reshape(1, -1), p['out_norm_b'].reshape(1, -1),
      p['out_proj_W'], p['out_proj_b'].reshape(1, -1))


def kernel(x, edge_index, cond, pos, params):
    p = params
    loops = jnp.arange(N, dtype=jnp.int32)
    src = jnp.concatenate([edge_index[0].astype(jnp.int32), loops])
    dst = jnp.concatenate([edge_index[1].astype(jnp.int32), loops])
    pad = NE_PAD - NE
    srcp = jnp.concatenate([src, jnp.zeros((pad,), jnp.int32)])
    dstp = jnp.concatenate([dst, jnp.full((pad,), jnp.int32(N))])

    if 2 not in _SC_CACHE:
        _SC_CACHE[2] = _make_sc_partition()
    srcs, dsts, cnts = _SC_CACHE[2](srcp, dstp)

    def seg_sum(z):
        o = _sc_spmm(HIDDEN, z, srcs, dsts, cnts)
        return jnp.concatenate(
            [o[0, :STRIDE], o[1, :N - STRIDE]], axis=0)

    ones128 = jnp.ones((N, HIDDEN), jnp.float32)
    deg = seg_sum(ones128)[:, 0:1]

    cond8 = jnp.pad(cond, ((0, 7), (0, 0)))
    gammas, betas, cps, cv = _prelude(cond8, p)

    pospad = jnp.pad(pos, ((0, 0), (0, HIDDEN - POS_DIM)))
    h = _input_proj(x, pospad, p, cv)

    def blk(i):
        return (p['blk%d_n1_g' % i].reshape(1, -1), p['blk%d_n1_b' % i].reshape(1, -1),
                p['blk%d_conv_W' % i], p['blk%d_conv_b' % i].reshape(1, -1),
                p['blk%d_n2_g' % i].reshape(1, -1), p['blk%d_n2_b' % i].reshape(1, -1),
                gammas[i:i + 1], betas[i:i + 1], cps[i:i + 1])

    masks = [jnp.ones((N, 1), jnp.float32)]
    skips = []
    off = 0
    n = N
    for d in range(DEPTH):
        n1g, n1b, cw, cb, n2g, n2b, gam, bet, cp = blk(off)
        zs = _pre(h, deg, masks[d], n1g, n1b, cp)
        acc = seg_sum(zs)
        h = _post(h, acc, deg, masks[d], cw, cb, n2g, n2b, gam, bet)
        off += 1
        skips.append(h)
        k = max(1, int(math.ceil(0.5 * n)))
        n = k
        s = _score(h, masks[d], p, d)
        masks.append(_topk_mask(s, k))

    n1g, n1b, cw, cb, n2g, n2b, gam, bet, cp = blk(off)
    zs = _pre(h, deg, masks[DEPTH], n1g, n1b, cp)
    acc = seg_sum(zs)
    h = _post(h, acc, deg, masks[DEPTH], cw, cb, n2g, n2b, gam, bet)
    off += 1

    for d in reversed(range(DEPTH)):
        n1g, n1b, cw, cb, n2g, n2b, gam, bet, cp = blk(off)
        heff, zs = _pre_unpool(h, skips[d], masks[d + 1], deg, masks[d],
                               n1g, n1b, cp)
        acc = seg_sum(zs)
        h = _post(heff, acc, deg, masks[d], cw, cb, n2g, n2b, gam, bet)
        off += 1

    return _final(h, p)
